# Initial kernel scaffold; baseline (speedup 1.0000x reference)
import jax
import jax.numpy as jnp
from jax.experimental import pallas as pl


def kernel(x, table):
    raise NotImplementedError("write your pallas kernel here")


# speedup vs baseline: 1.4949x; 1.4949x over previous
import functools

import jax
import jax.numpy as jnp
from jax import lax
from jax.experimental import pallas as pl
from jax.experimental.pallas import tpu as pltpu
from jax.experimental.pallas import tpu_sc as plsc

D = 32
NC = 2
NS = 16
NW = NC * NS


def _emb_body(chunk, nchunk, bpw, x_hbm, table_hbm, out_hbm, idx_v, rows_v, sem):
    wid = lax.axis_index("s") * NC + lax.axis_index("c")
    base = wid * bpw

    def body(i, carry):
        off = base + i * chunk
        pltpu.sync_copy(x_hbm.at[pl.ds(off, chunk)], idx_v)
        pltpu.async_copy(table_hbm.at[idx_v], rows_v, sem).wait()
        pltpu.sync_copy(rows_v, out_hbm.at[pl.ds(off, chunk)])
        return carry

    lax.fori_loop(0, nchunk, body, 0)


@jax.jit
def kernel(x, table):
    b = x.shape[0] * x.shape[1]
    bpw = b // NW
    chunk = 3200
    nchunk = bpw // chunk
    xf = x.reshape(b)
    mesh = plsc.VectorSubcoreMesh(core_axis_name="c", subcore_axis_name="s")
    fn = pl.kernel(
        functools.partial(_emb_body, chunk, nchunk, bpw),
        mesh=mesh,
        out_type=jax.ShapeDtypeStruct((b, D), jnp.float32),
        scratch_types=[
            pltpu.VMEM((chunk,), jnp.int32),
            pltpu.VMEM((chunk, D), jnp.float32),
            pltpu.SemaphoreType.DMA,
        ],
        compiler_params=pltpu.CompilerParams(use_tc_tiling_on_sc=False),
    )
    out = fn(xf, table)
    return out.reshape(x.shape + (D,))


# speedup vs baseline: 1.5003x; 1.0036x over previous
import functools

import jax
import jax.numpy as jnp
from jax import lax
from jax.experimental import pallas as pl
from jax.experimental.pallas import tpu as pltpu
from jax.experimental.pallas import tpu_sc as plsc

D = 32
NC = 2
NS = 16
NW = NC * NS
CHUNK = 800
NBUF = 4


def _emb_body(bpw, nchunk, x_hbm, table_hbm, out_hbm, idx_all, rows,
              sem_i, sg0, sg1, sg2, sg3, sw0, sw1, sw2, sw3):
    sg = (sg0, sg1, sg2, sg3)
    sw = (sw0, sw1, sw2, sw3)
    wid = lax.axis_index("s") * NC + lax.axis_index("c")
    base = wid * bpw

    pltpu.async_copy(x_hbm.at[pl.ds(base, bpw)], idx_all, sem_i).wait()

    def start_gather(i, b):
        pltpu.async_copy(
            table_hbm.at[idx_all.at[pl.ds(i * CHUNK, CHUNK)]], rows.at[b], sg[b])

    def wait_gather(b):
        pltpu.make_async_copy(
            table_hbm.at[idx_all.at[pl.ds(0, CHUNK)]], rows.at[b], sg[b]).wait()

    def start_wb(i, b):
        pltpu.async_copy(
            rows.at[b], out_hbm.at[pl.ds(base + i * CHUNK, CHUNK)], sw[b])

    def wait_wb(b):
        pltpu.make_async_copy(
            rows.at[b], out_hbm.at[pl.ds(0, CHUNK)], sw[b]).wait()

    start_gather(0, 0)
    start_gather(1, 1)
    for i in (0, 1):
        wait_gather(i)
        start_wb(i, i)
        start_gather(i + 2, i + 2)

    def steady(jj, carry):
        for k in range(NBUF):
            i = 2 + NBUF * jj + k
            b = (2 + k) % NBUF
            wait_gather(b)
            start_wb(i, b)
            b2 = (b + 2) % NBUF
            wait_wb(b2)
            start_gather(i + 2, b2)
        return carry

    lax.fori_loop(0, (nchunk - 4) // NBUF, steady, 0)

    for i in (nchunk - 2, nchunk - 1):
        b = i % NBUF
        wait_gather(b)
        start_wb(i, b)
        wait_wb((b + 2) % NBUF)
    for b in (2, 3):
        wait_wb(b)


@jax.jit
def kernel(x, table):
    b = x.shape[0] * x.shape[1]
    bpw = b // NW
    nchunk = bpw // CHUNK
    xf = x.reshape(b)
    mesh = plsc.VectorSubcoreMesh(core_axis_name="c", subcore_axis_name="s")
    fn = pl.kernel(
        functools.partial(_emb_body, bpw, nchunk),
        mesh=mesh,
        out_type=jax.ShapeDtypeStruct((b, D), jnp.float32),
        scratch_types=[
            pltpu.VMEM((bpw,), jnp.int32),
            pltpu.VMEM((NBUF, CHUNK, D), jnp.float32),
        ] + [pltpu.SemaphoreType.DMA] * 9,
        compiler_params=pltpu.CompilerParams(use_tc_tiling_on_sc=False),
    )
    out = fn(xf, table)
    return out.reshape(x.shape + (D,))


# speedup vs baseline: 1.5004x; 1.0000x over previous
import functools

import jax
import jax.numpy as jnp
from jax import lax
from jax.experimental import pallas as pl
from jax.experimental.pallas import tpu as pltpu
from jax.experimental.pallas import tpu_sc as plsc

D = 32
NC = 2
NS = 16
NW = NC * NS
NBUF = 8
G = 4


def _emb_body(rpw, x_hbm, table_hbm, out_hbm, idx_all, rows, sem_i, sg, sw):
    wid = lax.axis_index("s") * NC + lax.axis_index("c")
    rbase = wid * rpw

    pltpu.async_copy(x_hbm.at[pl.ds(rbase, rpw)], idx_all, sem_i).wait()

    def start_gather(i, b):
        pltpu.async_copy(table_hbm.at[idx_all.at[i]], rows.at[b], sg[b])

    def wait_gather(b):
        pltpu.make_async_copy(
            table_hbm.at[idx_all.at[0]], rows.at[b], sg[b]).wait()

    def start_wb(i, b):
        pltpu.async_copy(rows.at[b], out_hbm.at[rbase + i], sw[b])

    def wait_wb(b):
        pltpu.make_async_copy(rows.at[b], out_hbm.at[0], sw[b]).wait()

    for i in range(G):
        start_gather(i, i)
    for i in range(NBUF - G):
        wait_gather(i)
        start_wb(i, i)
        start_gather(i + G, i + G)
    h = NBUF - G

    def steady(jj, carry):
        for k in range(NBUF):
            i = h + NBUF * jj + k
            b = (h + k) % NBUF
            wait_gather(b)
            start_wb(i, b)
            b2 = (b + G) % NBUF
            wait_wb(b2)
            start_gather(i + G, b2)
        return carry

    lax.fori_loop(0, (rpw - h - G) // NBUF, steady, 0)

    for i in range(rpw - G, rpw):
        b = i % NBUF
        wait_gather(b)
        start_wb(i, b)
    for i in range(rpw - NBUF, rpw):
        wait_wb(i % NBUF)


@jax.jit
def kernel(x, table):
    batch, hist = x.shape
    rpw = batch // NW
    mesh = plsc.VectorSubcoreMesh(core_axis_name="c", subcore_axis_name="s")
    fn = pl.kernel(
        functools.partial(_emb_body, rpw),
        mesh=mesh,
        out_type=jax.ShapeDtypeStruct((batch, hist, D), jnp.float32),
        scratch_types=[
            pltpu.VMEM((rpw, hist), jnp.int32),
            pltpu.VMEM((NBUF, hist, D), jnp.float32),
            pltpu.SemaphoreType.DMA,
            [pltpu.SemaphoreType.DMA] * NBUF,
            [pltpu.SemaphoreType.DMA] * NBUF,
        ],
        compiler_params=pltpu.CompilerParams(use_tc_tiling_on_sc=False),
    )
    return fn(x, table)
